# baseline (speedup 1.0000x reference)
import jax
import jax.numpy as jnp
from jax import lax
from jax.experimental import pallas as pl
from jax.experimental.pallas import tpu as pltpu
from jax.experimental.pallas import tpu_sc as plsc

N = 10000
E = 320000
D = 128
NC = 2
NS = 16
NTILE = NC * NS
L = 16
BLK = 128
BPT = 80
B0 = 160
B1 = 0
HB = 16
EP = NTILE * BPT * BLK
NACC = N + 8
RPT = 624
RREM = N - NS * RPT
NP = N + L

_mesh = plsc.VectorSubcoreMesh(core_axis_name="c", subcore_axis_name="s",
                               num_cores=NC, num_subcores=NS)
_params = pltpu.CompilerParams(needs_layout_passes=False)


def _sc_agg_body(x_hbm, src_hbm, dst_hbm, out_hbm,
                 sidx, didx, rows0, rows1, acc, sem0, sem1):
    c = lax.axis_index("c")
    s = lax.axis_index("s")
    b0 = jnp.where(c == 0, s * B0, NS * B0 + s * B1)
    nch = jnp.where(c == 0, B0 // HB, B1 // HB)

    z = jnp.zeros((L,), jnp.float32)

    def zero_row(i, carry):
        for j in range(D // L):
            rows0[i, pl.ds(j * L, L)] = z
        return carry

    lax.fori_loop(0, BLK, zero_row, 0)

    r0 = s * RPT
    csz = [BLK] * (RPT // BLK) + [RPT % BLK]

    @pl.when(c == 0)
    def _zero_acc():
        for k, sz in enumerate(csz):
            pltpu.sync_copy(rows0.at[pl.ds(0, sz)], acc.at[pl.ds(r0 + k * BLK, sz)])

    @pl.when((s == NS - 1) & (c == 0))
    def _zero_tail():
        pltpu.sync_copy(rows0.at[pl.ds(0, RREM)], acc.at[pl.ds(NS * RPT, RREM)])

    plsc.subcore_barrier()

    def gather(b, buf, sem):
        pltpu.async_copy(x_hbm.at[sidx.at[b]], buf, sem)

    def wait_gather(buf, sem):
        pltpu.make_async_copy(x_hbm.at[pl.ds(0, BLK)], buf, sem).wait()

    def half_body(h, carry):
        pltpu.sync_copy(src_hbm.at[pl.ds(b0 + h * HB, HB)], sidx)
        pltpu.sync_copy(dst_hbm.at[pl.ds(b0 + h * HB, HB)], didx)
        gather(0, rows0, sem0)

        def pair(p, c2):
            b = 2 * p
            gather(b + 1, rows1, sem1)
            wait_gather(rows0, sem0)
            pltpu.sync_copy(rows0, acc.at[didx.at[b]], add=True)

            @pl.when(p < HB // 2 - 1)
            def _next():
                gather(b + 2, rows0, sem0)

            wait_gather(rows1, sem1)
            pltpu.sync_copy(rows1, acc.at[didx.at[b + 1]], add=True)
            return c2

        lax.fori_loop(0, HB // 2, pair, 0)
        return carry

    lax.fori_loop(0, nch, half_body, 0)
    plsc.subcore_barrier()

    @pl.when(c == 0)
    def _write_acc():
        for k, sz in enumerate(csz):
            pltpu.sync_copy(acc.at[pl.ds(r0 + k * BLK, sz)],
                            out_hbm.at[c, pl.ds(r0 + k * BLK, sz)])

    @pl.when((s == NS - 1) & (c == 0))
    def _write_tail():
        pltpu.sync_copy(acc.at[pl.ds(NS * RPT, RREM)],
                        out_hbm.at[c, pl.ds(NS * RPT, RREM)])


_sc_agg = pl.kernel(
    _sc_agg_body,
    out_type=jax.ShapeDtypeStruct((NC, N, D), jnp.float32),
    mesh=_mesh,
    scratch_types=[
        pltpu.VMEM((HB, BLK), jnp.int32),
        pltpu.VMEM((HB, BLK), jnp.int32),
        pltpu.VMEM((BLK, D), jnp.float32),
        pltpu.VMEM((BLK, D), jnp.float32),
        pltpu.VMEM_SHARED((NACC, D), jnp.float32),
        pltpu.SemaphoreType.DMA,
        pltpu.SemaphoreType.DMA,
    ],
    compiler_params=_params,
)


def _sc_deg_body(dst_hbm, deg_hbm, didx, dloc):
    c = lax.axis_index("c")
    s = lax.axis_index("s")
    w = s * NC + c
    b0 = w * BPT

    z = jnp.zeros((L,), jnp.float32)
    ones16 = jnp.ones((L,), jnp.float32)

    def zero_deg(i, carry):
        dloc[pl.ds(i * L, L)] = z
        return carry

    lax.fori_loop(0, NP // L, zero_deg, 0)

    def half_body(h, carry):
        pltpu.sync_copy(dst_hbm.at[pl.ds(b0 + h * HB, HB)], didx)

        def blk_body(b, c2):
            for k in range(BLK // L):
                dvec = didx[b, pl.ds(k * L, L)]
                plsc.addupdate_scatter(dloc, [dvec], ones16)
            return c2

        lax.fori_loop(0, HB, blk_body, 0)
        return carry

    lax.fori_loop(0, BPT // HB, half_body, 0)
    pltpu.sync_copy(dloc, deg_hbm.at[pl.ds(w * NP, NP)])


_sc_deg = pl.kernel(
    _sc_deg_body,
    out_type=jax.ShapeDtypeStruct((NTILE * NP,), jnp.float32),
    mesh=_mesh,
    scratch_types=[
        pltpu.VMEM((HB, BLK), jnp.int32),
        pltpu.VMEM((NP,), jnp.float32),
    ],
    compiler_params=_params,
)


def _tc1_body(agg_ref, dmat_ref, x_ref, wl_ref, wr_ref, b_ref, o_ref, dcol_ref):
    a = agg_ref[0]
    ones_nt = jnp.ones((NTILE, 1), jnp.float32)
    dcol = lax.dot_general(dmat_ref[...], ones_nt, (((0,), (0,)), ((), ())),
                           preferred_element_type=jnp.float32)
    deg = jnp.maximum(dcol, 1.0)
    dcol_ref[...] = deg
    m = a / deg
    o_ref[...] = jax.nn.relu(
        jnp.dot(m, wl_ref[...], preferred_element_type=jnp.float32)
        + b_ref[...]
        + jnp.dot(x_ref[...], wr_ref[...], preferred_element_type=jnp.float32))


def _tc2_body(agg_ref, dg_ref, x_ref, wl_ref, wr_ref, b_ref, o_ref):
    a = agg_ref[0]
    m = a / dg_ref[...]
    x2 = jax.nn.relu(
        jnp.dot(m, wl_ref[...], preferred_element_type=jnp.float32)
        + b_ref[...]
        + jnp.dot(x_ref[...], wr_ref[...], preferred_element_type=jnp.float32))
    r = lax.broadcasted_iota(jnp.int32, (D, 2 * D), 0)
    col = lax.broadcasted_iota(jnp.int32, (D, 2 * D), 1)
    p1 = (col == 2 * r).astype(jnp.float32)
    p2 = (col == 2 * r + 1).astype(jnp.float32)
    o_ref[...] = (jnp.dot(x_ref[...], p1, preferred_element_type=jnp.float32)
                  + jnp.dot(x2, p2, preferred_element_type=jnp.float32))


_tc1 = pl.pallas_call(
    _tc1_body,
    out_shape=[jax.ShapeDtypeStruct((N, D), jnp.float32),
               jax.ShapeDtypeStruct((N, 1), jnp.float32)],
)

_tc2 = pl.pallas_call(
    _tc2_body,
    out_shape=jax.ShapeDtypeStruct((N, 2 * D), jnp.float32),
)


def kernel(x, edge_index, W1l, W1r, b1, W2l, W2r, b2):
    ei = edge_index.astype(jnp.int32)
    src = jnp.concatenate([ei[0], jnp.zeros((EP - E,), jnp.int32)]).reshape(-1, BLK)
    dst = jnp.concatenate([ei[1], jnp.full((EP - E,), N, jnp.int32)]).reshape(-1, BLK)

    degflat = _sc_deg(dst)
    degmat = degflat.reshape(NTILE, NP)[:, :N]
    agg1 = _sc_agg(x, src, dst)
    x1, degcol = _tc1(agg1, degmat, x, W1l, W1r, b1.reshape(1, D))
    agg2 = _sc_agg(x1, src, dst)
    out256 = _tc2(agg2, degcol, x1, W2l, W2r, b2.reshape(1, D))
    return out256.reshape(N, D, 2)


# speedup vs baseline: 1.2632x; 1.2632x over previous
import jax
import jax.numpy as jnp
from jax import lax
from jax.experimental import pallas as pl
from jax.experimental.pallas import tpu as pltpu
from jax.experimental.pallas import tpu_sc as plsc

N = 10000
E = 320000
D = 128
NC = 2
NS = 16
NTILE = NC * NS
L = 16
BLK = 128
BPT = 80
B0 = 128
B1 = 32
HB = 16
EP = NTILE * BPT * BLK
NACC = N + 8
RPT = 624
RREM = N - NS * RPT
NP = N + L

_mesh = plsc.VectorSubcoreMesh(core_axis_name="c", subcore_axis_name="s",
                               num_cores=NC, num_subcores=NS)
_params = pltpu.CompilerParams(needs_layout_passes=False)


def _sc_agg_body(x_hbm, src_hbm, dst_hbm, out_hbm,
                 sidx, didx, rows0, rows1, acc, sem0, sem1):
    c = lax.axis_index("c")
    s = lax.axis_index("s")
    b0 = jnp.where(c == 0, s * B0, NS * B0 + s * B1)
    nch = jnp.where(c == 0, B0 // HB, B1 // HB)

    z = jnp.zeros((L,), jnp.float32)

    def zero_row(i, carry):
        for j in range(D // L):
            rows0[i, pl.ds(j * L, L)] = z
        return carry

    lax.fori_loop(0, BLK, zero_row, 0)

    r0 = s * RPT
    csz = [BLK] * (RPT // BLK) + [RPT % BLK]
    for k, sz in enumerate(csz):
        pltpu.sync_copy(rows0.at[pl.ds(0, sz)], acc.at[pl.ds(r0 + k * BLK, sz)])

    @pl.when(s == NS - 1)
    def _zero_tail():
        pltpu.sync_copy(rows0.at[pl.ds(0, RREM)], acc.at[pl.ds(NS * RPT, RREM)])

    plsc.subcore_barrier()

    def gather(b, buf, sem):
        pltpu.async_copy(x_hbm.at[sidx.at[b]], buf, sem)

    def wait_gather(buf, sem):
        pltpu.make_async_copy(x_hbm.at[pl.ds(0, BLK)], buf, sem).wait()

    def half_body(h, carry):
        pltpu.sync_copy(src_hbm.at[pl.ds(b0 + h * HB, HB)], sidx)
        pltpu.sync_copy(dst_hbm.at[pl.ds(b0 + h * HB, HB)], didx)
        gather(0, rows0, sem0)

        def pair(p, c2):
            b = 2 * p
            gather(b + 1, rows1, sem1)
            wait_gather(rows0, sem0)
            pltpu.sync_copy(rows0, acc.at[didx.at[b]], add=True)

            @pl.when(p < HB // 2 - 1)
            def _next():
                gather(b + 2, rows0, sem0)

            wait_gather(rows1, sem1)
            pltpu.sync_copy(rows1, acc.at[didx.at[b + 1]], add=True)
            return c2

        lax.fori_loop(0, HB // 2, pair, 0)
        return carry

    lax.fori_loop(0, nch, half_body, 0)
    plsc.subcore_barrier()

    for k, sz in enumerate(csz):
        pltpu.sync_copy(acc.at[pl.ds(r0 + k * BLK, sz)],
                        out_hbm.at[c, pl.ds(r0 + k * BLK, sz)])

    @pl.when(s == NS - 1)
    def _write_tail():
        pltpu.sync_copy(acc.at[pl.ds(NS * RPT, RREM)],
                        out_hbm.at[c, pl.ds(NS * RPT, RREM)])


_sc_agg = pl.kernel(
    _sc_agg_body,
    out_type=jax.ShapeDtypeStruct((NC, N, D), jnp.float32),
    mesh=_mesh,
    scratch_types=[
        pltpu.VMEM((HB, BLK), jnp.int32),
        pltpu.VMEM((HB, BLK), jnp.int32),
        pltpu.VMEM((BLK, D), jnp.float32),
        pltpu.VMEM((BLK, D), jnp.float32),
        pltpu.VMEM_SHARED((NACC, D), jnp.float32),
        pltpu.SemaphoreType.DMA,
        pltpu.SemaphoreType.DMA,
    ],
    compiler_params=_params,
)


def _sc_deg_body(dst_hbm, deg_hbm, didx, dloc):
    c = lax.axis_index("c")
    s = lax.axis_index("s")
    w = s * NC + c
    b0 = w * BPT

    z = jnp.zeros((L,), jnp.float32)
    ones16 = jnp.ones((L,), jnp.float32)

    def zero_deg(i, carry):
        dloc[pl.ds(i * L, L)] = z
        return carry

    lax.fori_loop(0, NP // L, zero_deg, 0)

    def half_body(h, carry):
        pltpu.sync_copy(dst_hbm.at[pl.ds(b0 + h * HB, HB)], didx)

        def blk_body(b, c2):
            for k in range(BLK // L):
                dvec = didx[b, pl.ds(k * L, L)]
                plsc.addupdate_scatter(dloc, [dvec], ones16)
            return c2

        lax.fori_loop(0, HB, blk_body, 0)
        return carry

    lax.fori_loop(0, BPT // HB, half_body, 0)
    pltpu.sync_copy(dloc, deg_hbm.at[pl.ds(w * NP, NP)])


_sc_deg = pl.kernel(
    _sc_deg_body,
    out_type=jax.ShapeDtypeStruct((NTILE * NP,), jnp.float32),
    mesh=_mesh,
    scratch_types=[
        pltpu.VMEM((HB, BLK), jnp.int32),
        pltpu.VMEM((NP,), jnp.float32),
    ],
    compiler_params=_params,
)


def _tc1_body(agg_ref, dmat_ref, x_ref, wl_ref, wr_ref, b_ref, o_ref, dcol_ref):
    a = agg_ref[0] + agg_ref[1]
    ones_nt = jnp.ones((NTILE, 1), jnp.float32)
    dcol = lax.dot_general(dmat_ref[...], ones_nt, (((0,), (0,)), ((), ())),
                           preferred_element_type=jnp.float32)
    deg = jnp.maximum(dcol, 1.0)
    dcol_ref[...] = deg
    m = a / deg
    o_ref[...] = jax.nn.relu(
        jnp.dot(m, wl_ref[...], preferred_element_type=jnp.float32)
        + b_ref[...]
        + jnp.dot(x_ref[...], wr_ref[...], preferred_element_type=jnp.float32))


def _tc2_body(agg_ref, dg_ref, x_ref, wl_ref, wr_ref, b_ref, o_ref):
    a = agg_ref[0] + agg_ref[1]
    m = a / dg_ref[...]
    o_ref[...] = jax.nn.relu(
        jnp.dot(m, wl_ref[...], preferred_element_type=jnp.float32)
        + b_ref[...]
        + jnp.dot(x_ref[...], wr_ref[...], preferred_element_type=jnp.float32))


_tc1 = pl.pallas_call(
    _tc1_body,
    out_shape=[jax.ShapeDtypeStruct((N, D), jnp.float32),
               jax.ShapeDtypeStruct((N, 1), jnp.float32)],
)

_tc2 = pl.pallas_call(
    _tc2_body,
    out_shape=jax.ShapeDtypeStruct((N, D), jnp.float32),
)


def kernel(x, edge_index, W1l, W1r, b1, W2l, W2r, b2):
    ei = edge_index.astype(jnp.int32)
    src = jnp.concatenate([ei[0], jnp.zeros((EP - E,), jnp.int32)]).reshape(-1, BLK)
    dst = jnp.concatenate([ei[1], jnp.full((EP - E,), N, jnp.int32)]).reshape(-1, BLK)

    degflat = _sc_deg(dst)
    degmat = degflat.reshape(NTILE, NP)[:, :N]
    agg1 = _sc_agg(x, src, dst)
    x1, degcol = _tc1(agg1, degmat, x, W1l, W1r, b1.reshape(1, D))
    agg2 = _sc_agg(x1, src, dst)
    x2 = _tc2(agg2, degcol, x1, W2l, W2r, b2.reshape(1, D))
    return jnp.stack([x1, x2], axis=2)
